# Initial kernel scaffold; baseline (speedup 1.0000x reference)
import functools

import jax
import jax.numpy as jnp
from jax import lax
from jax.experimental import pallas as pl
from jax.experimental.pallas import tpu as pltpu
from jax.experimental.pallas import tpu_sc as plsc

NUM_CLASSES = 1000
FEATURE_DIM = 512
BATCH = 16384

NUM_WORKERS = 32
ROWS_PER_WORKER = BATCH // NUM_WORKERS
CHUNK = 64
NCHUNKS = ROWS_PER_WORKER // CHUNK
ACC_ROWS = 1024
AUX_W = 16


def _sc_phase1(input_hbm, tgt_hbm, center_hbm, out_hbm,
               xbuf, cbuf, aux, tgt, acc_sh, sem):
    cid = lax.axis_index("c")
    sid = lax.axis_index("s")
    ncores = lax.axis_size("c")
    wid = sid * ncores + cid

    lane = lax.iota(jnp.int32, 16)
    zeros16 = jnp.zeros((16,), jnp.float32)

    def zero_body(r, _):
        aux[r, :] = zeros16
        return 0
    lax.fori_loop(0, CHUNK, zero_body, 0)
    rows_per_sub = ACC_ROWS // 16
    pltpu.sync_copy(aux, acc_sh.at[pl.ds(sid * rows_per_sub, rows_per_sub)])

    pltpu.sync_copy(tgt_hbm.at[wid], tgt)
    plsc.subcore_barrier()

    for j in range(NCHUNKS):
        base = wid * ROWS_PER_WORKER + j * CHUNK
        pltpu.sync_copy(input_hbm.at[pl.ds(base, CHUNK)], xbuf)
        pltpu.async_copy(center_hbm.at[tgt.at[j]], cbuf, sem).wait()

        def row_body(r, _):
            acc = zeros16
            for i in range(FEATURE_DIM // 16):
                xv = xbuf[r, pl.ds(i * 16, 16)]
                cv = cbuf[r, pl.ds(i * 16, 16)]
                d = xv - cv
                acc = acc + d * d
            sq = jnp.sum(acc)
            vec = jnp.where(lane == 0, sq,
                            jnp.where(lane == 1, 1.0, 0.0)).astype(jnp.float32)
            aux[r, :] = vec
            return 0
        lax.fori_loop(0, CHUNK, row_body, 0)

        pltpu.sync_copy(aux, acc_sh.at[tgt.at[j]], add=True)

    plsc.subcore_barrier()
    pltpu.sync_copy(acc_sh.at[pl.ds(sid * rows_per_sub, rows_per_sub)],
                    out_hbm.at[cid, pl.ds(sid * rows_per_sub, rows_per_sub)])


def _tc_epilogue(acc_ref, lam_ref, o_ref):
    w = acc_ref[0] + acc_ref[1]
    lane = lax.broadcasted_iota(jnp.int32, (ACC_ROWS, AUX_W), 1)
    s = jnp.where(lane == 0, w, 0.0)
    cnt = jnp.sum(jnp.where(lane == 1, w, 0.0), axis=1, keepdims=True)
    ratio = jnp.where(cnt > 0, s / jnp.where(cnt > 0, cnt, 1.0), 0.0)
    o_ref[0, 0] = jnp.sum(ratio) * lam_ref[0] * (0.5 / BATCH)


def kernel(input, target, lambdas, center):
    tgt3 = target.astype(jnp.int32).reshape(NUM_WORKERS, NCHUNKS, CHUNK)

    mesh = plsc.VectorSubcoreMesh(core_axis_name="c", subcore_axis_name="s")
    num_cores = 2
    acc = pl.kernel(
        _sc_phase1,
        out_type=jax.ShapeDtypeStruct((num_cores, ACC_ROWS, AUX_W),
                                      jnp.float32),
        mesh=mesh,
        scratch_types=[
            pltpu.VMEM((CHUNK, FEATURE_DIM), jnp.float32),
            pltpu.VMEM((CHUNK, FEATURE_DIM), jnp.float32),
            pltpu.VMEM((CHUNK, AUX_W), jnp.float32),
            pltpu.VMEM((NCHUNKS, CHUNK), jnp.int32),
            pltpu.VMEM_SHARED((ACC_ROWS, AUX_W), jnp.float32),
            pltpu.SemaphoreType.DMA,
        ],
    )(input, tgt3, center)

    lam = jnp.asarray(lambdas, jnp.float32).reshape(1)
    out = pl.pallas_call(
        _tc_epilogue,
        out_shape=jax.ShapeDtypeStruct((1, 1), jnp.float32),
        in_specs=[
            pl.BlockSpec(memory_space=pltpu.VMEM),
            pl.BlockSpec(memory_space=pltpu.SMEM),
        ],
        out_specs=pl.BlockSpec(memory_space=pltpu.VMEM),
    )(acc, lam)
    return out[0, 0]


# speedup vs baseline: 2.2712x; 2.2712x over previous
import jax
import jax.numpy as jnp
from jax import lax
from jax.experimental import pallas as pl
from jax.experimental.pallas import tpu as pltpu
from jax.experimental.pallas import tpu_sc as plsc

NUM_CLASSES = 1000
FEATURE_DIM = 512
BATCH = 16384

NCORES = 2
NUM_WORKERS = 32
ROWS_PER_WORKER = BATCH // NUM_WORKERS
CHUNK = 64
NCHUNKS = ROWS_PER_WORKER // CHUNK
ACC_ROWS = 1024
AUX_W = 128


def _sc_phase1(input_hbm, tgt_hbm, center_hbm, out_hbm,
               xbuf, cbuf, aux, tgt, acc_sh, sem):
    cid = lax.axis_index("c")
    sid = lax.axis_index("s")
    wid = sid * NCORES + cid

    lane = lax.iota(jnp.int32, 16)
    zeros16 = jnp.zeros((16,), jnp.float32)

    def zero_body(r, _):
        for i in range(AUX_W // 16):
            aux[r, pl.ds(i * 16, 16)] = zeros16
        return 0
    lax.fori_loop(0, CHUNK, zero_body, 0)
    rows_per_sub = ACC_ROWS // 16
    pltpu.sync_copy(aux, acc_sh.at[pl.ds(sid * rows_per_sub, rows_per_sub)])

    pltpu.sync_copy(tgt_hbm.at[wid], tgt)
    plsc.subcore_barrier()

    for j in range(NCHUNKS):
        base = wid * ROWS_PER_WORKER + j * CHUNK
        pltpu.sync_copy(input_hbm.at[pl.ds(base, CHUNK)], xbuf)
        pltpu.async_copy(center_hbm.at[tgt.at[j]], cbuf, sem).wait()

        def row_body(r, _):
            acc = zeros16
            for i in range(FEATURE_DIM // 16):
                xv = xbuf[r, pl.ds(i * 16, 16)]
                cv = cbuf[r, pl.ds(i * 16, 16)]
                d = xv - cv
                acc = acc + d * d
            sq = jnp.sum(acc)
            vec = jnp.where(lane == 0, sq,
                            jnp.where(lane == 1, 1.0, 0.0)).astype(jnp.float32)
            aux[r, pl.ds(0, 16)] = vec
            return 0
        lax.fori_loop(0, CHUNK, row_body, 0)

        pltpu.sync_copy(aux, acc_sh.at[tgt.at[j]], add=True)

    plsc.subcore_barrier()
    pltpu.sync_copy(acc_sh.at[pl.ds(sid * rows_per_sub, rows_per_sub)],
                    out_hbm.at[cid, pl.ds(sid * rows_per_sub, rows_per_sub)])


def _tc_epilogue(acc_ref, lam_ref, o_ref):
    w = acc_ref[0] + acc_ref[1]
    lane = lax.broadcasted_iota(jnp.int32, (ACC_ROWS, AUX_W), 1)
    s = jnp.where(lane == 0, w, 0.0)
    cnt = jnp.sum(jnp.where(lane == 1, w, 0.0), axis=1, keepdims=True)
    ratio = jnp.where(cnt > 0, s / jnp.where(cnt > 0, cnt, 1.0), 0.0)
    val = jnp.sum(ratio) * lam_ref[0] * (0.5 / BATCH)
    o_ref[...] = jnp.full((1, 1), val, jnp.float32)


def kernel(input, target, lambdas, center):
    tgt3 = target.astype(jnp.int32).reshape(NUM_WORKERS, NCHUNKS, CHUNK)

    mesh = plsc.VectorSubcoreMesh(core_axis_name="c", subcore_axis_name="s")
    acc = pl.kernel(
        _sc_phase1,
        out_type=jax.ShapeDtypeStruct((NCORES, ACC_ROWS, AUX_W),
                                      jnp.float32),
        mesh=mesh,
        compiler_params=pltpu.CompilerParams(needs_layout_passes=False),
        scratch_types=[
            pltpu.VMEM((CHUNK, FEATURE_DIM), jnp.float32),
            pltpu.VMEM((CHUNK, FEATURE_DIM), jnp.float32),
            pltpu.VMEM((CHUNK, AUX_W), jnp.float32),
            pltpu.VMEM((NCHUNKS, CHUNK), jnp.int32),
            pltpu.VMEM_SHARED((ACC_ROWS, AUX_W), jnp.float32),
            pltpu.SemaphoreType.DMA,
        ],
    )(input, tgt3, center)

    lam = jnp.asarray(lambdas, jnp.float32).reshape(1)
    out = pl.pallas_call(
        _tc_epilogue,
        out_shape=jax.ShapeDtypeStruct((1, 1), jnp.float32),
        in_specs=[
            pl.BlockSpec(memory_space=pltpu.VMEM),
            pl.BlockSpec(memory_space=pltpu.SMEM),
        ],
        out_specs=pl.BlockSpec(memory_space=pltpu.VMEM),
    )(acc, lam)
    return out[0, 0]


# speedup vs baseline: 3.1977x; 1.4080x over previous
import jax
import jax.numpy as jnp
from jax import lax
from jax.experimental import pallas as pl
from jax.experimental.pallas import tpu as pltpu
from jax.experimental.pallas import tpu_sc as plsc

NUM_CLASSES = 1000
FEATURE_DIM = 512
BATCH = 16384

NCORES = 2
NUM_WORKERS = 32
ROWS_PER_WORKER = BATCH // NUM_WORKERS
CHUNK = 32
NCHUNKS = ROWS_PER_WORKER // CHUNK
ACC_ROWS = 1024
AUX_W = 128


def _sc_phase1(input_hbm, tgt_hbm, center_hbm, out_hbm,
               xbuf0, xbuf1, cbuf0, cbuf1, aux0, aux1, tgt, acc_sh,
               isem0, isem1, gsem0, gsem1, ssem0, ssem1):
    cid = lax.axis_index("c")
    sid = lax.axis_index("s")
    wid = sid * NCORES + cid

    xbufs, cbufs, auxs = (xbuf0, xbuf1), (cbuf0, cbuf1), (aux0, aux1)
    isems, gsems, ssems = (isem0, isem1), (gsem0, gsem1), (ssem0, ssem1)

    lane = lax.iota(jnp.int32, 16)
    zeros16 = jnp.zeros((16,), jnp.float32)

    def zero_body(r, _):
        for i in range(AUX_W // 16):
            aux0[r, pl.ds(i * 16, 16)] = zeros16
            aux1[r, pl.ds(i * 16, 16)] = zeros16
        return 0
    lax.fori_loop(0, CHUNK, zero_body, 0)
    rows_per_sub = ACC_ROWS // 16
    for h in range(rows_per_sub // CHUNK):
        pltpu.sync_copy(
            aux0, acc_sh.at[pl.ds(sid * rows_per_sub + h * CHUNK, CHUNK)])

    pltpu.sync_copy(tgt_hbm.at[wid], tgt)
    plsc.subcore_barrier()

    def start_in(j, b):
        base = wid * ROWS_PER_WORKER + j * CHUNK
        return pltpu.async_copy(input_hbm.at[pl.ds(base, CHUNK)],
                                xbufs[b], isems[b])

    def start_gather(j, b):
        return pltpu.async_copy(center_hbm.at[tgt.at[j]], cbufs[b], gsems[b])

    pend = {}
    pend[0] = (start_in(0, 0), start_gather(0, 0))
    scat = {}
    for j in range(NCHUNKS):
        b = j % 2
        if j + 1 < NCHUNKS:
            pend[j + 1] = (start_in(j + 1, 1 - b), start_gather(j + 1, 1 - b))
        din, dg = pend.pop(j)
        din.wait()
        dg.wait()
        if j >= 2:
            scat.pop(j - 2).wait()
        xbuf, cbuf, aux = xbufs[b], cbufs[b], auxs[b]

        def row_body(r, _):
            acc = zeros16
            for i in range(FEATURE_DIM // 16):
                xv = xbuf[r, pl.ds(i * 16, 16)]
                cv = cbuf[r, pl.ds(i * 16, 16)]
                d = xv - cv
                acc = acc + d * d
            sq = jnp.sum(acc)
            vec = jnp.where(lane == 0, sq,
                            jnp.where(lane == 1, 1.0, 0.0)).astype(jnp.float32)
            aux[r, pl.ds(0, 16)] = vec
            return 0
        lax.fori_loop(0, CHUNK, row_body, 0)

        scat[j] = pltpu.async_copy(aux, acc_sh.at[tgt.at[j]], ssems[b],
                                   add=True)

    scat.pop(NCHUNKS - 2).wait()
    scat.pop(NCHUNKS - 1).wait()
    plsc.subcore_barrier()
    pltpu.sync_copy(acc_sh.at[pl.ds(sid * rows_per_sub, rows_per_sub)],
                    out_hbm.at[cid, pl.ds(sid * rows_per_sub, rows_per_sub)])


def _tc_epilogue(acc_ref, lam_ref, o_ref):
    w = acc_ref[0] + acc_ref[1]
    lane = lax.broadcasted_iota(jnp.int32, (ACC_ROWS, AUX_W), 1)
    s = jnp.where(lane == 0, w, 0.0)
    cnt = jnp.sum(jnp.where(lane == 1, w, 0.0), axis=1, keepdims=True)
    ratio = jnp.where(cnt > 0, s / jnp.where(cnt > 0, cnt, 1.0), 0.0)
    val = jnp.sum(ratio) * lam_ref[0] * (0.5 / BATCH)
    o_ref[...] = jnp.full((1, 1), val, jnp.float32)


def kernel(input, target, lambdas, center):
    tgt3 = target.astype(jnp.int32).reshape(NUM_WORKERS, NCHUNKS, CHUNK)

    mesh = plsc.VectorSubcoreMesh(core_axis_name="c", subcore_axis_name="s")
    acc = pl.kernel(
        _sc_phase1,
        out_type=jax.ShapeDtypeStruct((NCORES, ACC_ROWS, AUX_W),
                                      jnp.float32),
        mesh=mesh,
        compiler_params=pltpu.CompilerParams(needs_layout_passes=False),
        scratch_types=[
            pltpu.VMEM((CHUNK, FEATURE_DIM), jnp.float32),
            pltpu.VMEM((CHUNK, FEATURE_DIM), jnp.float32),
            pltpu.VMEM((CHUNK, FEATURE_DIM), jnp.float32),
            pltpu.VMEM((CHUNK, FEATURE_DIM), jnp.float32),
            pltpu.VMEM((CHUNK, AUX_W), jnp.float32),
            pltpu.VMEM((CHUNK, AUX_W), jnp.float32),
            pltpu.VMEM((NCHUNKS, CHUNK), jnp.int32),
            pltpu.VMEM_SHARED((ACC_ROWS, AUX_W), jnp.float32),
            pltpu.SemaphoreType.DMA,
            pltpu.SemaphoreType.DMA,
            pltpu.SemaphoreType.DMA,
            pltpu.SemaphoreType.DMA,
            pltpu.SemaphoreType.DMA,
            pltpu.SemaphoreType.DMA,
        ],
    )(input, tgt3, center)

    lam = jnp.asarray(lambdas, jnp.float32).reshape(1)
    out = pl.pallas_call(
        _tc_epilogue,
        out_shape=jax.ShapeDtypeStruct((1, 1), jnp.float32),
        in_specs=[
            pl.BlockSpec(memory_space=pltpu.VMEM),
            pl.BlockSpec(memory_space=pltpu.SMEM),
        ],
        out_specs=pl.BlockSpec(memory_space=pltpu.VMEM),
    )(acc, lam)
    return out[0, 0]
